# baseline (speedup 1.0000x reference)
import functools
import math

import jax
import jax.numpy as jnp
from jax import lax
from jax.experimental import pallas as pl
from jax.experimental.pallas import tpu as pltpu
from jax.experimental.pallas import tpu_sc as plsc

D = 64
SCALE = 8.0

NC = 2
NS = 16
NW = NC * NS

LANES = 16
SBLK = 128
NBUF = 4
PREFETCH = 2


def _sc_embed_native(xT, lut2):
    T, S = xT.shape
    assert S == NW * SBLK
    assert T % NBUF == 0

    mesh = plsc.VectorSubcoreMesh(core_axis_name="c", subcore_axis_name="s")

    @functools.partial(
        pl.kernel,
        out_type=jax.ShapeDtypeStruct((T, D, S), jnp.float32),
        mesh=mesh,
        scratch_types=[
            pltpu.VMEM((T, SBLK), jnp.int32),
            pltpu.VMEM((NBUF, SBLK), jnp.int32),
            pltpu.VMEM((NBUF, SBLK, 2 * D), jnp.float32),
            pltpu.VMEM((2, D, SBLK), jnp.float32),
            pltpu.SemaphoreType.DMA((NBUF,)),
            pltpu.SemaphoreType.DMA((2,)),
        ],
        compiler_params=pltpu.CompilerParams(needs_layout_passes=False),
    )
    def k(x_hbm, tab_hbm, y_hbm, idx_v, pidx_v, g_v, out_v, gsem, osem):
        wid = lax.axis_index("s") * NC + lax.axis_index("c")
        s0 = pl.multiple_of(wid * SBLK, SBLK)
        pltpu.sync_copy(x_hbm.at[:, pl.ds(s0, SBLK)], idx_v)

        iota = lax.iota(jnp.int32, LANES)

        def prep_and_fire(t, b):
            for q in range(SBLK // LANES):
                sl = pl.ds(q * LANES, LANES)
                pidx_v[b, sl] = lax.shift_right_logical(idx_v[t, sl], 1)
            pltpu.async_copy(
                tab_hbm.at[pidx_v.at[b]], g_v.at[b], gsem.at[b]
            )

        def wait_gather(b):
            pltpu.make_async_copy(
                tab_hbm.at[pl.ds(0, SBLK)], g_v.at[b], gsem.at[b]
            ).wait()

        def wait_out(ob):
            pltpu.make_async_copy(
                out_v.at[ob], y_hbm.at[0, :, pl.ds(0, SBLK)], osem.at[ob]
            ).wait()

        for p in range(PREFETCH):
            prep_and_fire(p, p)

        def super_body(sidx, carry):
            t0 = sidx * NBUF
            for j in range(NBUF):
                t = t0 + j
                ob = j % 2
                f = t + PREFETCH

                @pl.when(f < T)
                def _():
                    prep_and_fire(f, (j + PREFETCH) % NBUF)

                wait_gather(j)

                @pl.when(t >= 2)
                def _():
                    wait_out(ob)

                for q in range(SBLK // LANES):
                    sl = pl.ds(q * LANES, LANES)
                    iv = idx_v[t, sl]
                    row = iota + (q * LANES)
                    col0 = lax.shift_left(
                        lax.bitwise_and(iv, jnp.int32(1)), jnp.int32(6)
                    )

                    @plsc.parallel_loop(0, D, 1, unroll=8)
                    def _(d):
                        vals = plsc.load_gather(g_v.at[j], [row, col0 + d])
                        out_v[ob, d, sl] = vals * SCALE

                pltpu.async_copy(
                    out_v.at[ob],
                    y_hbm.at[t, :, pl.ds(s0, SBLK)],
                    osem.at[ob],
                )
            return carry

        lax.fori_loop(0, T // NBUF, super_body, 0)

        for ob in range(2):
            wait_out(ob)

    return k(xT, lut2)


def kernel(x, lut):
    S, T = x.shape
    xT = x.T
    lut2 = jnp.concatenate([lut[0::2], lut[1::2]], axis=1)
    y = _sc_embed_native(xT, lut2)
    return jnp.transpose(y, (2, 0, 1))


# speedup vs baseline: 7.3793x; 7.3793x over previous
import functools
import math

import jax
import jax.numpy as jnp
from jax import lax
from jax.experimental import pallas as pl
from jax.experimental.pallas import tpu as pltpu
from jax.experimental.pallas import tpu_sc as plsc

D = 64
SCALE = 8.0

NC = 2
NS = 16
NW = NC * NS

LANES = 16
SBLK = 128
NBUF = 4
PREFETCH = 2


def _sc_embed_native(xT, lut2):
    T, S = xT.shape
    assert S == NW * SBLK
    assert T % NBUF == 0

    mesh = plsc.VectorSubcoreMesh(core_axis_name="c", subcore_axis_name="s")

    @functools.partial(
        pl.kernel,
        out_type=jax.ShapeDtypeStruct((T, D, S), jnp.float32),
        mesh=mesh,
        scratch_types=[
            pltpu.VMEM((T, SBLK), jnp.int32),
            pltpu.VMEM((NBUF, SBLK), jnp.int32),
            pltpu.VMEM((NBUF, SBLK, 2 * D), jnp.float32),
            pltpu.VMEM((2, D, SBLK), jnp.float32),
            pltpu.SemaphoreType.DMA((NBUF,)),
            pltpu.SemaphoreType.DMA((2,)),
        ],
        compiler_params=pltpu.CompilerParams(needs_layout_passes=False),
    )
    def k(x_hbm, tab_hbm, y_hbm, idx_v, pidx_v, g_v, out_v, gsem, osem):
        wid = lax.axis_index("s") * NC + lax.axis_index("c")
        s0 = pl.multiple_of(wid * SBLK, SBLK)
        pltpu.sync_copy(x_hbm.at[:, pl.ds(s0, SBLK)], idx_v)

        iota = lax.iota(jnp.int32, LANES)

        def prep_and_fire(t, b):
            for q in range(SBLK // LANES):
                sl = pl.ds(q * LANES, LANES)
                pidx_v[b, sl] = lax.shift_right_logical(idx_v[t, sl], 1)
            pltpu.async_copy(
                tab_hbm.at[pidx_v.at[b]], g_v.at[b], gsem.at[b]
            )

        def wait_gather(b):
            pltpu.make_async_copy(
                tab_hbm.at[pl.ds(0, SBLK)], g_v.at[b], gsem.at[b]
            ).wait()

        def wait_out(ob):
            pltpu.make_async_copy(
                out_v.at[ob], y_hbm.at[0, :, pl.ds(0, SBLK)], osem.at[ob]
            ).wait()

        for p in range(PREFETCH):
            prep_and_fire(p, p)

        def super_body(sidx, carry):
            t0 = sidx * NBUF
            for j in range(NBUF):
                t = t0 + j
                ob = j % 2
                f = t + PREFETCH

                @pl.when(f < T)
                def _():
                    prep_and_fire(f, (j + PREFETCH) % NBUF)

                wait_gather(j)

                @pl.when(t >= 2)
                def _():
                    wait_out(ob)

                for q in range(SBLK // LANES):
                    sl = pl.ds(q * LANES, LANES)
                    iv = idx_v[t, sl]
                    row = iota + (q * LANES)
                    col0 = lax.shift_left(
                        lax.bitwise_and(iv, jnp.int32(1)), jnp.int32(6)
                    )

                    @plsc.parallel_loop(0, D, 1, unroll=8)
                    def _(d):
                        vals = plsc.load_gather(g_v.at[j], [row, col0 + d])
                        out_v[ob, d, sl] = vals * SCALE

                pltpu.async_copy(
                    out_v.at[ob],
                    y_hbm.at[t, :, pl.ds(s0, SBLK)],
                    osem.at[ob],
                )
            return carry

        lax.fori_loop(0, T // NBUF, super_body, 0)

        for ob in range(2):
            wait_out(ob)

    return k(xT, lut2)


def kernel(x, lut):
    S, T = x.shape
    xT = x.T
    lut2 = lut.reshape(-1, 2 * D)
    y = _sc_embed_native(xT, lut2)
    return jnp.transpose(y, (2, 0, 1))
